# baseline (speedup 1.0000x reference)
import functools

import jax
import jax.numpy as jnp
from jax import lax
from jax.experimental import pallas as pl
from jax.experimental.pallas import tpu as pltpu
from jax.experimental.pallas import tpu_sc as plsc

B, L, HIDDEN = 4096, 200, 128
N = B * L
NC, NS = 2, 16
NW = NC * NS
LANES = 16


def _build(n_tokens, hidden, k_chunk, interpret=False):
    tpw = n_tokens // NW
    chunks = tpw // k_chunk
    ncol = hidden // LANES
    mesh = plsc.VectorSubcoreMesh(
        core_axis_name="c", subcore_axis_name="s", num_cores=NC, num_subcores=NS
    )

    @functools.partial(
        pl.kernel,
        out_type=jax.ShapeDtypeStruct((n_tokens, hidden), jnp.float32),
        mesh=mesh,
        scratch_types=[
            pltpu.VMEM((k_chunk,), jnp.int32),
            pltpu.VMEM((k_chunk,), jnp.int32),
            pltpu.VMEM((k_chunk,), jnp.int32),
            pltpu.VMEM((k_chunk, hidden), jnp.float32),
            pltpu.VMEM((k_chunk, hidden), jnp.float32),
            pltpu.VMEM((k_chunk, hidden), jnp.float32),
            pltpu.SemaphoreType.DMA,
            pltpu.SemaphoreType.DMA,
            pltpu.SemaphoreType.DMA,
        ],
        interpret=interpret,
    )
    def sc_embed(ids_hbm, pos_hbm, seg_hbm, wt_hbm, pt_hbm, st_hbm, out_hbm,
                 idw, idp, idg, wb, pb, sb, semw, semp, semg):
        wid = lax.axis_index("s") * NC + lax.axis_index("c")
        base0 = wid * tpw

        def chunk_body(i, carry):
            base = base0 + i * k_chunk
            pltpu.sync_copy(ids_hbm.at[pl.ds(base, k_chunk)], idw)
            pltpu.sync_copy(pos_hbm.at[pl.ds(base, k_chunk)], idp)
            pltpu.sync_copy(seg_hbm.at[pl.ds(base, k_chunk)], idg)
            cw = pltpu.async_copy(wt_hbm.at[idw], wb, semw)
            cp = pltpu.async_copy(pt_hbm.at[idp], pb, semp)
            cg = pltpu.async_copy(st_hbm.at[idg], sb, semg)
            cw.wait()
            cp.wait()
            cg.wait()

            def tok_body(t, c2):
                for j in range(ncol):
                    sl = pl.ds(j * LANES, LANES)
                    wb[t, sl] = wb[t, sl] + pb[t, sl] + sb[t, sl]
                return c2

            lax.fori_loop(0, k_chunk, tok_body, 0, unroll=False)
            pltpu.sync_copy(wb, out_hbm.at[pl.ds(base, k_chunk)])
            return carry

        lax.fori_loop(0, chunks, chunk_body, 0, unroll=False)

    return sc_embed


def kernel(input_ids, position_ids, token_type_ids, word_table, pos_table, seg_table):
    ids = input_ids.reshape(N).astype(jnp.int32)
    pos = position_ids.reshape(N).astype(jnp.int32)
    seg = token_type_ids.reshape(N).astype(jnp.int32)
    fn = _build(N, HIDDEN, 128)
    out = fn(ids, pos, seg, word_table, pos_table, seg_table)
    return out.reshape(B, L, HIDDEN)
